# baseline (speedup 1.0000x reference)
import functools

import jax
import jax.numpy as jnp
from jax.experimental import pallas as pl


def _edge_body(h_ref, ea_ref, src_ref, dst_ref, We_ref, be_ref, aggr_ref, *, npad):
    i = pl.program_id(0)

    @pl.when(i == 0)
    def _init():
        aggr_ref[...] = jnp.zeros_like(aggr_ref)

    e = jnp.dot(ea_ref[...], We_ref[...], preferred_element_type=jnp.float32)
    e = e + be_ref[...]
    src = src_ref[0, 0, :]
    dst = dst_ref[0, 0, :]
    eb = src.shape[0]
    ids = jax.lax.broadcasted_iota(jnp.int32, (eb, npad), 1)
    oh_src = (src[:, None] == ids).astype(jnp.float32)
    hs = jnp.dot(oh_src, h_ref[...], preferred_element_type=jnp.float32,
                 precision=jax.lax.Precision.HIGHEST)
    msg = jax.nn.relu(hs + e)
    oh_dst = (dst[:, None] == ids).astype(jnp.float32)
    aggr_ref[...] += jax.lax.dot_general(
        oh_dst, msg, (((0,), (0,)), ((), ())), preferred_element_type=jnp.float32,
        precision=jax.lax.Precision.HIGHEST)


def _edge_stage(h_pad, edge_attr, src3, dst3, We, be, npad, d, e_blk):
    n_blk = src3.shape[0]
    body = functools.partial(_edge_body, npad=npad)
    return pl.pallas_call(
        body,
        grid=(n_blk,),
        in_specs=[
            pl.BlockSpec((npad, d), lambda i: (0, 0)),
            pl.BlockSpec((e_blk, We.shape[0]), lambda i: (i, 0)),
            pl.BlockSpec((1, 1, e_blk), lambda i: (i, 0, 0)),
            pl.BlockSpec((1, 1, e_blk), lambda i: (i, 0, 0)),
            pl.BlockSpec(We.shape, lambda i: (0, 0)),
            pl.BlockSpec((1, d), lambda i: (0, 0)),
        ],
        out_specs=pl.BlockSpec((npad, d), lambda i: (0, 0)),
        out_shape=jax.ShapeDtypeStruct((npad, d), jnp.float32),
    )(h_pad, edge_attr, src3, dst3, We, be)


def _node_body(h_ref, aggr_ref, W1_ref, b1_ref, g1_ref, be1_ref, W2_ref, b2_ref,
               gbn_ref, bbn_ref, eps_ref, out_ref, *, final):
    pre = (1.0 + eps_ref[0, 0]) * h_ref[...] + aggr_ref[...]
    z = jnp.dot(pre, W1_ref[...], preferred_element_type=jnp.float32) + b1_ref[...]
    m = jnp.mean(z, axis=0, keepdims=True)
    v = jnp.mean((z - m) ** 2, axis=0, keepdims=True)
    z = (z - m) * jax.lax.rsqrt(v + 1e-5) * g1_ref[...] + be1_ref[...]
    z = jax.nn.relu(z)
    hn = jnp.dot(z, W2_ref[...], preferred_element_type=jnp.float32) + b2_ref[...]
    m2 = jnp.mean(hn, axis=0, keepdims=True)
    v2 = jnp.mean((hn - m2) ** 2, axis=0, keepdims=True)
    hn = (hn - m2) * jax.lax.rsqrt(v2 + 1e-5) * gbn_ref[...] + bbn_ref[...]
    if not final:
        hn = jax.nn.relu(hn)
    out_ref[...] = hn


def _node_stage(h, aggr, W1, b1, g1, be1, W2, b2, gbn, bbn, eps, final):
    n, d = h.shape
    body = functools.partial(_node_body, final=final)
    return pl.pallas_call(
        body,
        out_shape=jax.ShapeDtypeStruct((n, d), jnp.float32),
    )(h, aggr, W1, b1, g1, be1, W2, b2, gbn, bbn, eps)


def kernel(x, edge_index, edge_attr, W_edge, b_edge, W1, b1, g1, be1, W2, b2,
           g_bn, b_bn, eps_gin):
    n, d = x.shape
    n_edges = edge_attr.shape[0]
    n_layers = W_edge.shape[0]
    e_blk = 128
    assert n_edges % e_blk == 0
    npad = ((n + 127) // 128) * 128

    src3 = edge_index[0].reshape(n_edges // e_blk, 1, e_blk)
    dst3 = edge_index[1].reshape(n_edges // e_blk, 1, e_blk)

    h = x
    for l in range(n_layers):
        h_pad = jnp.pad(h, ((0, npad - n), (0, 0)))
        aggr = _edge_stage(h_pad, edge_attr, src3, dst3,
                           W_edge[l], b_edge[l].reshape(1, d), npad, d, e_blk)
        aggr = aggr[:n]
        h = _node_stage(h, aggr,
                        W1[l], b1[l].reshape(1, -1), g1[l].reshape(1, -1),
                        be1[l].reshape(1, -1), W2[l], b2[l].reshape(1, d),
                        g_bn[l].reshape(1, d), b_bn[l].reshape(1, d),
                        eps_gin[l].reshape(1, 1), final=(l == n_layers - 1))
    return h


# speedup vs baseline: 32.7917x; 32.7917x over previous
import functools

import jax
import jax.numpy as jnp
from jax import lax
from jax.experimental import pallas as pl
from jax.experimental.pallas import tpu as pltpu
from jax.experimental.pallas import tpu_sc as plsc

N_WORKERS = 32
CHUNK = 80
E_BLK_TC = 6400


def _e_body(ea_ref, We_ref, be_ref, out_ref):
    out_ref[0] = jnp.dot(ea_ref[...], We_ref[0],
                         preferred_element_type=jnp.float32) + be_ref[0]


def _e_stage(edge_attr, W_edge, b_edge):
    n_layers, de, d = W_edge.shape
    n_edges = edge_attr.shape[0]
    n_blk = n_edges // E_BLK_TC
    return pl.pallas_call(
        _e_body,
        grid=(n_layers, n_blk),
        in_specs=[
            pl.BlockSpec((E_BLK_TC, de), lambda l, j: (j, 0)),
            pl.BlockSpec((1, de, d), lambda l, j: (l, 0, 0)),
            pl.BlockSpec((1, 1, d), lambda l, j: (l, 0, 0)),
        ],
        out_specs=pl.BlockSpec((1, E_BLK_TC, d), lambda l, j: (l, j, 0)),
        out_shape=jax.ShapeDtypeStruct((n_layers, n_edges, d), jnp.float32),
    )(edge_attr, W_edge, b_edge.reshape(n_layers, 1, d))


def _make_sc_edge(n, d, n_edges):
    epw = n_edges // N_WORKERS
    n_chunks = epw // CHUNK
    n_pad = -(-n // (16 * CHUNK)) * (16 * CHUNK)
    stripe = n_pad // 16
    nvec = d // 16
    mesh = plsc.VectorSubcoreMesh(core_axis_name="c", subcore_axis_name="s")

    @functools.partial(
        pl.kernel,
        mesh=mesh,
        out_type=jax.ShapeDtypeStruct((2, n_pad, d), jnp.float32),
        scratch_types=[
            pltpu.VMEM((CHUNK,), jnp.int32),
            pltpu.VMEM((CHUNK,), jnp.int32),
            pltpu.VMEM((CHUNK, d), jnp.float32),
            pltpu.VMEM((CHUNK, d), jnp.float32),
            pltpu.VMEM_SHARED((n_pad, d), jnp.float32),
            pltpu.SemaphoreType.DMA,
            pltpu.SemaphoreType.DMA,
        ],
    )
    def sc_edge(h_hbm, e_hbm, src_hbm, dst_hbm, out_hbm,
                src_v, dst_v, rows_v, e_v, aggr_sh, sem, sem2):
        c = lax.axis_index("c")
        s = lax.axis_index("s")
        wid = s * 2 + c
        base = wid * epw

        def zrow(k, _):
            for cc in range(nvec):
                rows_v[k, pl.ds(cc * 16, 16)] = jnp.zeros((16,), jnp.float32)
            return 0
        lax.fori_loop(0, CHUNK, zrow, 0)
        row0 = s * stripe
        for t in range(stripe // CHUNK):
            pltpu.sync_copy(rows_v, aggr_sh.at[pl.ds(row0 + t * CHUNK, CHUNK)])
        plsc.subcore_barrier()

        def chunk(j, _):
            pltpu.sync_copy(src_hbm.at[wid, j], src_v)
            pltpu.sync_copy(dst_hbm.at[wid, j], dst_v)
            cp = pltpu.async_copy(h_hbm.at[src_v], rows_v, sem)
            pltpu.sync_copy(e_hbm.at[pl.ds(base + j * CHUNK, CHUNK)], e_v)
            cp.wait()

            def row(k, _):
                for cc in range(nvec):
                    sl = pl.ds(cc * 16, 16)
                    rows_v[k, sl] = jnp.maximum(rows_v[k, sl] + e_v[k, sl], 0.0)
                return 0
            lax.fori_loop(0, CHUNK, row, 0)
            pltpu.async_copy(rows_v, aggr_sh.at[dst_v], sem2, add=True).wait()
            return 0
        lax.fori_loop(0, n_chunks, chunk, 0)

        plsc.subcore_barrier()
        pltpu.sync_copy(aggr_sh.at[pl.ds(row0, stripe)],
                        out_hbm.at[c, pl.ds(row0, stripe)])

    return sc_edge


def _node_body(h_ref, a0_ref, a1_ref, W1_ref, b1_ref, g1_ref, be1_ref, W2_ref,
               b2_ref, gbn_ref, bbn_ref, eps_ref, out_ref, *, final):
    pre = (1.0 + eps_ref[0, 0]) * h_ref[...] + a0_ref[...] + a1_ref[...]
    z = jnp.dot(pre, W1_ref[...], preferred_element_type=jnp.float32) + b1_ref[...]
    m = jnp.mean(z, axis=0, keepdims=True)
    v = jnp.mean((z - m) ** 2, axis=0, keepdims=True)
    z = (z - m) * lax.rsqrt(v + 1e-5) * g1_ref[...] + be1_ref[...]
    z = jax.nn.relu(z)
    hn = jnp.dot(z, W2_ref[...], preferred_element_type=jnp.float32) + b2_ref[...]
    m2 = jnp.mean(hn, axis=0, keepdims=True)
    v2 = jnp.mean((hn - m2) ** 2, axis=0, keepdims=True)
    hn = (hn - m2) * lax.rsqrt(v2 + 1e-5) * gbn_ref[...] + bbn_ref[...]
    if not final:
        hn = jax.nn.relu(hn)
    out_ref[...] = hn


def _node_stage(h, a0, a1, W1, b1, g1, be1, W2, b2, gbn, bbn, eps, final):
    n, d = h.shape
    body = functools.partial(_node_body, final=final)
    return pl.pallas_call(
        body,
        out_shape=jax.ShapeDtypeStruct((n, d), jnp.float32),
    )(h, a0, a1, W1, b1, g1, be1, W2, b2, gbn, bbn, eps)


def kernel(x, edge_index, edge_attr, W_edge, b_edge, W1, b1, g1, be1, W2, b2,
           g_bn, b_bn, eps_gin):
    n, d = x.shape
    n_edges = edge_attr.shape[0]
    n_layers = W_edge.shape[0]
    epw = n_edges // N_WORKERS
    assert n_edges % N_WORKERS == 0 and epw % CHUNK == 0 and n % 16 == 0

    e_all = _e_stage(edge_attr, W_edge, b_edge)
    src3 = edge_index[0].reshape(N_WORKERS, epw // CHUNK, CHUNK)
    dst3 = edge_index[1].reshape(N_WORKERS, epw // CHUNK, CHUNK)
    sc_edge = _make_sc_edge(n, d, n_edges)

    h = x
    for l in range(n_layers):
        parts = sc_edge(h, e_all[l], src3, dst3)
        h = _node_stage(h, parts[0, :n], parts[1, :n],
                        W1[l], b1[l].reshape(1, -1), g1[l].reshape(1, -1),
                        be1[l].reshape(1, -1), W2[l], b2[l].reshape(1, d),
                        g_bn[l].reshape(1, d), b_bn[l].reshape(1, d),
                        eps_gin[l].reshape(1, 1), final=(l == n_layers - 1))
    return h


# speedup vs baseline: 45.2034x; 1.3785x over previous
import functools

import jax
import jax.numpy as jnp
from jax import lax
from jax.experimental import pallas as pl
from jax.experimental.pallas import tpu as pltpu
from jax.experimental.pallas import tpu_sc as plsc

N_WORKERS = 32
CHUNK = 80
E_BLK_TC = 6400


def _e_body(ea_ref, We_ref, be_ref, out_ref):
    out_ref[0] = jnp.dot(ea_ref[...], We_ref[0],
                         preferred_element_type=jnp.float32) + be_ref[0]


def _e_stage(edge_attr, W_edge, b_edge):
    n_layers, de, d = W_edge.shape
    n_edges = edge_attr.shape[0]
    n_blk = n_edges // E_BLK_TC
    return pl.pallas_call(
        _e_body,
        grid=(n_layers, n_blk),
        in_specs=[
            pl.BlockSpec((E_BLK_TC, de), lambda l, j: (j, 0)),
            pl.BlockSpec((1, de, d), lambda l, j: (l, 0, 0)),
            pl.BlockSpec((1, 1, d), lambda l, j: (l, 0, 0)),
        ],
        out_specs=pl.BlockSpec((1, E_BLK_TC, d), lambda l, j: (l, j, 0)),
        out_shape=jax.ShapeDtypeStruct((n_layers, n_edges, d), jnp.float32),
    )(edge_attr, W_edge, b_edge.reshape(n_layers, 1, d))


def _make_sc_edge(n, d, n_edges):
    epw = n_edges // N_WORKERS
    n_chunks = epw // CHUNK
    assert n_chunks % 2 == 1 and n_chunks >= 3
    n_pairs = n_chunks // 2
    n_pad = -(-n // (16 * CHUNK)) * (16 * CHUNK)
    stripe = n_pad // 16
    nvec = d // 16
    mesh = plsc.VectorSubcoreMesh(core_axis_name="c", subcore_axis_name="s")

    @functools.partial(
        pl.kernel,
        mesh=mesh,
        out_type=jax.ShapeDtypeStruct((2, n_pad, d), jnp.float32),
        scratch_types=[
            pltpu.VMEM((2, 2, CHUNK), jnp.int32),
            pltpu.VMEM((2, CHUNK, d), jnp.float32),
            pltpu.VMEM_SHARED((n_pad, d), jnp.float32),
            pltpu.SemaphoreType.DMA,
            pltpu.SemaphoreType.DMA,
            pltpu.SemaphoreType.DMA,
            pltpu.SemaphoreType.DMA,
            pltpu.SemaphoreType.DMA,
            pltpu.SemaphoreType.DMA,
            pltpu.SemaphoreType.DMA,
            pltpu.SemaphoreType.DMA,
        ],
    )
    def sc_edge(h_hbm, e_hbm, sd_hbm, out_hbm, id_v, rows_v, aggr_sh,
                g0, g1, s0, s1, i0, i1, e0, e1):
        c = lax.axis_index("c")
        s = lax.axis_index("s")
        wid = s * 2 + c
        base = wid * epw
        gsem = (g0, g1)
        ssem = (s0, s1)
        isem = (i0, i1)
        esem = (e0, e1)

        def zrow(k, _):
            for cc in range(nvec):
                rows_v[0, k, pl.ds(cc * 16, 16)] = jnp.zeros((16,), jnp.float32)
            return 0
        lax.fori_loop(0, CHUNK, zrow, 0)
        row0 = s * stripe
        for t in range(stripe // CHUNK):
            pltpu.sync_copy(rows_v.at[0], aggr_sh.at[pl.ds(row0 + t * CHUNK, CHUNK)])
        plsc.subcore_barrier()

        def ids_start(j, b):
            return pltpu.async_copy(sd_hbm.at[wid, j], id_v.at[b], isem[b])

        def e_start(j, b):
            return pltpu.async_copy(e_hbm.at[pl.ds(base + j * CHUNK, CHUNK)],
                                    rows_v.at[b], esem[b])

        def gather_start(b):
            return pltpu.async_copy(h_hbm.at[id_v.at[b, 0]], rows_v.at[b],
                                    gsem[b], add=True)

        def scatter_start(b):
            return pltpu.async_copy(rows_v.at[b], aggr_sh.at[id_v.at[b, 1]],
                                    ssem[b], add=True)

        def relu(b):
            def row(k, _):
                for cc in range(nvec):
                    sl = pl.ds(cc * 16, 16)
                    rows_v[b, k, sl] = jnp.maximum(rows_v[b, k, sl], 0.0)
                return 0
            lax.fori_loop(0, CHUNK, row, 0)

        def wait(cp):
            cp.wait()

        wait(ids_start(0, 0))
        wait(e_start(0, 0))
        cp_g0 = gather_start(0)
        cp_i1 = ids_start(1, 1)
        cp_e1 = e_start(1, 1)

        def pair_body(p, _):
            c0 = 2 * p
            c1 = c0 + 1
            pltpu.make_async_copy(h_hbm.at[id_v.at[0, 0]], rows_v.at[0], gsem[0]).wait()
            relu(0)
            scatter_start(0)
            pltpu.make_async_copy(sd_hbm.at[wid, c1], id_v.at[1], isem[1]).wait()
            pltpu.make_async_copy(e_hbm.at[pl.ds(base + c1 * CHUNK, CHUNK)],
                                  rows_v.at[1], esem[1]).wait()
            gather_start(1)
            pltpu.make_async_copy(rows_v.at[0], aggr_sh.at[id_v.at[0, 1]], ssem[0]).wait()
            ids_start(c0 + 2, 0)
            e_start(c0 + 2, 0)
            pltpu.make_async_copy(h_hbm.at[id_v.at[1, 0]], rows_v.at[1], gsem[1]).wait()
            relu(1)
            scatter_start(1)
            pltpu.make_async_copy(rows_v.at[1], aggr_sh.at[id_v.at[1, 1]], ssem[1]).wait()
            c1n = jnp.minimum(c1 + 2, n_chunks - 1)
            ids_start(c1n, 1)
            e_start(c1n, 1)
            pltpu.make_async_copy(sd_hbm.at[wid, c0 + 2], id_v.at[0], isem[0]).wait()
            pltpu.make_async_copy(e_hbm.at[pl.ds(base + (c0 + 2) * CHUNK, CHUNK)],
                                  rows_v.at[0], esem[0]).wait()
            gather_start(0)
            return 0
        lax.fori_loop(0, n_pairs, pair_body, 0)

        pltpu.make_async_copy(h_hbm.at[id_v.at[0, 0]], rows_v.at[0], gsem[0]).wait()
        relu(0)
        scatter_start(0)
        pltpu.make_async_copy(rows_v.at[0], aggr_sh.at[id_v.at[0, 1]], ssem[0]).wait()
        pltpu.make_async_copy(sd_hbm.at[wid, 0], id_v.at[1], isem[1]).wait()
        pltpu.make_async_copy(e_hbm.at[pl.ds(base, CHUNK)], rows_v.at[1], esem[1]).wait()

        plsc.subcore_barrier()
        pltpu.sync_copy(aggr_sh.at[pl.ds(row0, stripe)],
                        out_hbm.at[c, pl.ds(row0, stripe)])

    return sc_edge


def _node_body(h_ref, a0_ref, a1_ref, W1_ref, b1_ref, g1_ref, be1_ref, W2_ref,
               b2_ref, gbn_ref, bbn_ref, eps_ref, out_ref, *, final):
    pre = (1.0 + eps_ref[0, 0]) * h_ref[...] + a0_ref[...] + a1_ref[...]
    z = jnp.dot(pre, W1_ref[...], preferred_element_type=jnp.float32) + b1_ref[...]
    m = jnp.mean(z, axis=0, keepdims=True)
    v = jnp.mean((z - m) ** 2, axis=0, keepdims=True)
    z = (z - m) * lax.rsqrt(v + 1e-5) * g1_ref[...] + be1_ref[...]
    z = jax.nn.relu(z)
    hn = jnp.dot(z, W2_ref[...], preferred_element_type=jnp.float32) + b2_ref[...]
    m2 = jnp.mean(hn, axis=0, keepdims=True)
    v2 = jnp.mean((hn - m2) ** 2, axis=0, keepdims=True)
    hn = (hn - m2) * lax.rsqrt(v2 + 1e-5) * gbn_ref[...] + bbn_ref[...]
    if not final:
        hn = jax.nn.relu(hn)
    out_ref[...] = hn


def _node_stage(h, a0, a1, W1, b1, g1, be1, W2, b2, gbn, bbn, eps, final):
    n, d = h.shape
    body = functools.partial(_node_body, final=final)
    return pl.pallas_call(
        body,
        out_shape=jax.ShapeDtypeStruct((n, d), jnp.float32),
    )(h, a0, a1, W1, b1, g1, be1, W2, b2, gbn, bbn, eps)


def kernel(x, edge_index, edge_attr, W_edge, b_edge, W1, b1, g1, be1, W2, b2,
           g_bn, b_bn, eps_gin):
    n, d = x.shape
    n_edges = edge_attr.shape[0]
    n_layers = W_edge.shape[0]
    epw = n_edges // N_WORKERS
    assert n_edges % N_WORKERS == 0 and epw % CHUNK == 0 and n % 16 == 0

    e_all = _e_stage(edge_attr, W_edge, b_edge)
    src3 = edge_index[0].reshape(N_WORKERS, epw // CHUNK, CHUNK)
    dst3 = edge_index[1].reshape(N_WORKERS, epw // CHUNK, CHUNK)
    sd3 = jnp.stack([src3, dst3], axis=2)
    sc_edge = _make_sc_edge(n, d, n_edges)

    h = x
    for l in range(n_layers):
        parts = sc_edge(h, e_all[l], sd3)
        h = _node_stage(h, parts[0, :n], parts[1, :n],
                        W1[l], b1[l].reshape(1, -1), g1[l].reshape(1, -1),
                        be1[l].reshape(1, -1), W2[l], b2[l].reshape(1, d),
                        g_bn[l].reshape(1, d), b_bn[l].reshape(1, d),
                        eps_gin[l].reshape(1, 1), final=(l == n_layers - 1))
    return h


# speedup vs baseline: 61.9543x; 1.3706x over previous
import functools

import jax
import jax.numpy as jnp
from jax import lax
from jax.experimental import pallas as pl
from jax.experimental.pallas import tpu as pltpu
from jax.experimental.pallas import tpu_sc as plsc

N_WORKERS = 32
CHUNK = 80
E_BLK_TC = 6400


def _e_body(ea_ref, We_ref, be_ref, out_ref):
    out_ref[...] = jnp.dot(ea_ref[...], We_ref[...],
                           preferred_element_type=jnp.float32) + be_ref[...]


def _e_stage(edge_attr, We, be):
    de, d = We.shape
    n_edges = edge_attr.shape[0]
    n_blk = n_edges // E_BLK_TC
    return pl.pallas_call(
        _e_body,
        grid=(n_blk,),
        in_specs=[
            pl.BlockSpec((E_BLK_TC, de), lambda j: (j, 0)),
            pl.BlockSpec((de, d), lambda j: (0, 0)),
            pl.BlockSpec((1, d), lambda j: (0, 0)),
        ],
        out_specs=pl.BlockSpec((E_BLK_TC, d), lambda j: (j, 0)),
        out_shape=jax.ShapeDtypeStruct((n_edges, d), jnp.float32),
    )(edge_attr, We, be.reshape(1, d))


def _make_sc_edge(n, d, n_edges):
    epw = n_edges // N_WORKERS
    n_chunks = epw // CHUNK
    assert n_chunks % 2 == 1 and n_chunks >= 3
    n_pairs = n_chunks // 2
    n_pad = -(-n // (16 * CHUNK)) * (16 * CHUNK)
    stripe = n_pad // 16
    nvec = d // 16
    mesh = plsc.VectorSubcoreMesh(core_axis_name="c", subcore_axis_name="s")

    @functools.partial(
        pl.kernel,
        mesh=mesh,
        out_type=jax.ShapeDtypeStruct((2, n_pad, d), jnp.float32),
        scratch_types=[
            pltpu.VMEM((2, 2, CHUNK), jnp.int32),
            pltpu.VMEM((2, CHUNK, d), jnp.float32),
            pltpu.VMEM_SHARED((n_pad, d), jnp.float32),
            pltpu.SemaphoreType.DMA,
            pltpu.SemaphoreType.DMA,
            pltpu.SemaphoreType.DMA,
            pltpu.SemaphoreType.DMA,
            pltpu.SemaphoreType.DMA,
            pltpu.SemaphoreType.DMA,
            pltpu.SemaphoreType.DMA,
            pltpu.SemaphoreType.DMA,
        ],
    )
    def sc_edge(h_hbm, e_hbm, sd_hbm, out_hbm, id_v, rows_v, aggr_sh,
                g0, g1, s0, s1, i0, i1, e0, e1):
        c = lax.axis_index("c")
        s = lax.axis_index("s")
        wid = s * 2 + c
        base = wid * epw
        gsem = (g0, g1)
        ssem = (s0, s1)
        isem = (i0, i1)
        esem = (e0, e1)

        def zrow(k, _):
            for cc in range(nvec):
                rows_v[0, k, pl.ds(cc * 16, 16)] = jnp.zeros((16,), jnp.float32)
            return 0
        lax.fori_loop(0, CHUNK, zrow, 0)
        row0 = s * stripe
        for t in range(stripe // CHUNK):
            pltpu.sync_copy(rows_v.at[0], aggr_sh.at[pl.ds(row0 + t * CHUNK, CHUNK)])
        plsc.subcore_barrier()

        def ids_start(j, b):
            return pltpu.async_copy(sd_hbm.at[wid, j], id_v.at[b], isem[b])

        def e_start(j, b):
            return pltpu.async_copy(e_hbm.at[pl.ds(base + j * CHUNK, CHUNK)],
                                    rows_v.at[b], esem[b])

        def gather_start(b):
            return pltpu.async_copy(h_hbm.at[id_v.at[b, 0]], rows_v.at[b],
                                    gsem[b], add=True)

        def scatter_start(b):
            return pltpu.async_copy(rows_v.at[b], aggr_sh.at[id_v.at[b, 1]],
                                    ssem[b], add=True)

        def relu(b):
            def row(k, _):
                for cc in range(nvec):
                    sl = pl.ds(cc * 16, 16)
                    rows_v[b, k, sl] = jnp.maximum(rows_v[b, k, sl], 0.0)
                return 0
            lax.fori_loop(0, CHUNK, row, 0)

        def wait(cp):
            cp.wait()

        wait(ids_start(0, 0))
        wait(e_start(0, 0))
        cp_g0 = gather_start(0)
        cp_i1 = ids_start(1, 1)
        cp_e1 = e_start(1, 1)

        def pair_body(p, _):
            c0 = 2 * p
            c1 = c0 + 1
            pltpu.make_async_copy(h_hbm.at[id_v.at[0, 0]], rows_v.at[0], gsem[0]).wait()
            relu(0)
            scatter_start(0)
            pltpu.make_async_copy(sd_hbm.at[wid, c1], id_v.at[1], isem[1]).wait()
            pltpu.make_async_copy(e_hbm.at[pl.ds(base + c1 * CHUNK, CHUNK)],
                                  rows_v.at[1], esem[1]).wait()
            gather_start(1)
            pltpu.make_async_copy(rows_v.at[0], aggr_sh.at[id_v.at[0, 1]], ssem[0]).wait()
            ids_start(c0 + 2, 0)
            e_start(c0 + 2, 0)
            pltpu.make_async_copy(h_hbm.at[id_v.at[1, 0]], rows_v.at[1], gsem[1]).wait()
            relu(1)
            scatter_start(1)
            pltpu.make_async_copy(rows_v.at[1], aggr_sh.at[id_v.at[1, 1]], ssem[1]).wait()
            c1n = jnp.minimum(c1 + 2, n_chunks - 1)
            ids_start(c1n, 1)
            e_start(c1n, 1)
            pltpu.make_async_copy(sd_hbm.at[wid, c0 + 2], id_v.at[0], isem[0]).wait()
            pltpu.make_async_copy(e_hbm.at[pl.ds(base + (c0 + 2) * CHUNK, CHUNK)],
                                  rows_v.at[0], esem[0]).wait()
            gather_start(0)
            return 0
        lax.fori_loop(0, n_pairs, pair_body, 0)

        pltpu.make_async_copy(h_hbm.at[id_v.at[0, 0]], rows_v.at[0], gsem[0]).wait()
        relu(0)
        scatter_start(0)
        pltpu.make_async_copy(rows_v.at[0], aggr_sh.at[id_v.at[0, 1]], ssem[0]).wait()
        pltpu.make_async_copy(sd_hbm.at[wid, 0], id_v.at[1], isem[1]).wait()
        pltpu.make_async_copy(e_hbm.at[pl.ds(base, CHUNK)], rows_v.at[1], esem[1]).wait()

        plsc.subcore_barrier()
        pltpu.sync_copy(aggr_sh.at[pl.ds(row0, stripe)],
                        out_hbm.at[c, pl.ds(row0, stripe)])

    return sc_edge


def _node_body(h_ref, parts_ref, W1_ref, b1_ref, g1_ref, be1_ref, W2_ref,
               b2_ref, gbn_ref, bbn_ref, eps_ref, out_ref, *, final):
    pre = (1.0 + eps_ref[0, 0]) * h_ref[...] + parts_ref[0] + parts_ref[1]
    z = jnp.dot(pre, W1_ref[...], preferred_element_type=jnp.float32) + b1_ref[...]
    m = jnp.mean(z, axis=0, keepdims=True)
    v = jnp.mean((z - m) ** 2, axis=0, keepdims=True)
    z = (z - m) * lax.rsqrt(v + 1e-5) * g1_ref[...] + be1_ref[...]
    z = jax.nn.relu(z)
    hn = jnp.dot(z, W2_ref[...], preferred_element_type=jnp.float32) + b2_ref[...]
    m2 = jnp.mean(hn, axis=0, keepdims=True)
    v2 = jnp.mean((hn - m2) ** 2, axis=0, keepdims=True)
    hn = (hn - m2) * lax.rsqrt(v2 + 1e-5) * gbn_ref[...] + bbn_ref[...]
    if not final:
        hn = jax.nn.relu(hn)
    out_ref[...] = hn


def _node_stage(h, parts, W1, b1, g1, be1, W2, b2, gbn, bbn, eps, final):
    n, d = h.shape
    hdim = W1.shape[1]
    body = functools.partial(_node_body, final=final)
    return pl.pallas_call(
        body,
        grid=(1,),
        in_specs=[
            pl.BlockSpec((n, d), lambda i: (0, 0)),
            pl.BlockSpec((2, n, d), lambda i: (0, 0, 0)),
            pl.BlockSpec((d, hdim), lambda i: (0, 0)),
            pl.BlockSpec((1, hdim), lambda i: (0, 0)),
            pl.BlockSpec((1, hdim), lambda i: (0, 0)),
            pl.BlockSpec((1, hdim), lambda i: (0, 0)),
            pl.BlockSpec((hdim, d), lambda i: (0, 0)),
            pl.BlockSpec((1, d), lambda i: (0, 0)),
            pl.BlockSpec((1, d), lambda i: (0, 0)),
            pl.BlockSpec((1, d), lambda i: (0, 0)),
            pl.BlockSpec((1, 1), lambda i: (0, 0)),
        ],
        out_specs=pl.BlockSpec((n, d), lambda i: (0, 0)),
        out_shape=jax.ShapeDtypeStruct((n, d), jnp.float32),
    )(h, parts, W1, b1, g1, be1, W2, b2, gbn, bbn, eps)


def kernel(x, edge_index, edge_attr, W_edge, b_edge, W1, b1, g1, be1, W2, b2,
           g_bn, b_bn, eps_gin):
    n, d = x.shape
    n_edges = edge_attr.shape[0]
    n_layers = W_edge.shape[0]
    epw = n_edges // N_WORKERS
    assert n_edges % N_WORKERS == 0 and epw % CHUNK == 0 and n % 16 == 0

    src3 = edge_index[0].reshape(N_WORKERS, epw // CHUNK, CHUNK)
    dst3 = edge_index[1].reshape(N_WORKERS, epw // CHUNK, CHUNK)
    sd3 = jnp.stack([src3, dst3], axis=2)
    sc_edge = _make_sc_edge(n, d, n_edges)

    h = x
    for l in range(n_layers):
        e_l = _e_stage(edge_attr, W_edge[l], b_edge[l])
        parts = sc_edge(h, e_l, sd3)
        h = _node_stage(h, parts,
                        W1[l], b1[l].reshape(1, -1), g1[l].reshape(1, -1),
                        be1[l].reshape(1, -1), W2[l], b2[l].reshape(1, d),
                        g_bn[l].reshape(1, d), b_bn[l].reshape(1, d),
                        eps_gin[l].reshape(1, 1), final=(l == n_layers - 1))
    return h


# speedup vs baseline: 64.8368x; 1.0465x over previous
import functools

import jax
import jax.numpy as jnp
from jax import lax
from jax.experimental import pallas as pl
from jax.experimental.pallas import tpu as pltpu
from jax.experimental.pallas import tpu_sc as plsc

N_WORKERS = 32
CHUNK = 80
E_BLK_TC = 6400


def _e_body(ea_ref, We_ref, be_ref, out_ref):
    out_ref[...] = jnp.dot(ea_ref[...], We_ref[...],
                           preferred_element_type=jnp.float32) + be_ref[...]


def _e_stage(edge_attr, We, be):
    de, d = We.shape
    n_edges = edge_attr.shape[0]
    n_blk = n_edges // E_BLK_TC
    return pl.pallas_call(
        _e_body,
        grid=(n_blk,),
        in_specs=[
            pl.BlockSpec((E_BLK_TC, de), lambda j: (j, 0)),
            pl.BlockSpec((de, d), lambda j: (0, 0)),
            pl.BlockSpec((1, d), lambda j: (0, 0)),
        ],
        out_specs=pl.BlockSpec((E_BLK_TC, d), lambda j: (j, 0)),
        out_shape=jax.ShapeDtypeStruct((n_edges, d), jnp.float32),
    )(edge_attr, We, be.reshape(1, d))


def _make_sc_edge(n, d, n_edges):
    epw = n_edges // N_WORKERS
    n_chunks = epw // CHUNK
    assert n_chunks % 2 == 1 and n_chunks >= 3
    n_pairs = n_chunks // 2
    n_pad = -(-n // (16 * CHUNK)) * (16 * CHUNK)
    stripe = n_pad // 16
    nvec = d // 16
    mesh = plsc.VectorSubcoreMesh(core_axis_name="c", subcore_axis_name="s")

    @functools.partial(
        pl.kernel,
        mesh=mesh,
        out_type=jax.ShapeDtypeStruct((2, n_pad, d), jnp.float32),
        scratch_types=[
            pltpu.VMEM((2, 2, CHUNK), jnp.int32),
            pltpu.VMEM((2, CHUNK, d), jnp.float32),
            pltpu.VMEM_SHARED((n_pad, d), jnp.float32),
            pltpu.SemaphoreType.DMA,
            pltpu.SemaphoreType.DMA,
            pltpu.SemaphoreType.DMA,
            pltpu.SemaphoreType.DMA,
            pltpu.SemaphoreType.DMA,
            pltpu.SemaphoreType.DMA,
            pltpu.SemaphoreType.DMA,
            pltpu.SemaphoreType.DMA,
        ],
    )
    def sc_edge(h_hbm, e_hbm, sd_hbm, out_hbm, id_v, rows_v, aggr_sh,
                g0, g1, s0, s1, i0, i1, e0, e1):
        c = lax.axis_index("c")
        s = lax.axis_index("s")
        wid = s * 2 + c
        base = wid * epw
        gsem = (g0, g1)
        ssem = (s0, s1)
        isem = (i0, i1)
        esem = (e0, e1)

        def zrow(k, _):
            for cc in range(nvec):
                rows_v[0, k, pl.ds(cc * 16, 16)] = jnp.zeros((16,), jnp.float32)
            return 0
        lax.fori_loop(0, CHUNK, zrow, 0)
        row0 = s * stripe
        for t in range(stripe // CHUNK):
            pltpu.sync_copy(rows_v.at[0], aggr_sh.at[pl.ds(row0 + t * CHUNK, CHUNK)])
        plsc.subcore_barrier()

        def ids_start(j, b):
            return pltpu.async_copy(sd_hbm.at[wid, j], id_v.at[b], isem[b])

        def e_start(j, b):
            return pltpu.async_copy(e_hbm.at[pl.ds(base + j * CHUNK, CHUNK)],
                                    rows_v.at[b], esem[b])

        def gather_start(b):
            return pltpu.async_copy(h_hbm.at[id_v.at[b, 0]], rows_v.at[b],
                                    gsem[b], add=True)

        def scatter_start(b):
            return pltpu.async_copy(rows_v.at[b], aggr_sh.at[id_v.at[b, 1]],
                                    ssem[b], add=True)

        def relu(b):
            def row(k, _):
                for cc in range(nvec):
                    sl = pl.ds(cc * 16, 16)
                    rows_v[b, k, sl] = jnp.maximum(rows_v[b, k, sl], 0.0)
                return 0
            lax.fori_loop(0, CHUNK, row, 0)

        def wait(cp):
            cp.wait()

        wait(ids_start(0, 0))
        wait(e_start(0, 0))
        cp_g0 = gather_start(0)
        cp_i1 = ids_start(1, 1)
        cp_e1 = e_start(1, 1)

        def pair_body(p, _):
            c0 = 2 * p
            c1 = c0 + 1
            pltpu.make_async_copy(h_hbm.at[id_v.at[0, 0]], rows_v.at[0], gsem[0]).wait()
            relu(0)
            scatter_start(0)
            pltpu.make_async_copy(sd_hbm.at[wid, c1], id_v.at[1], isem[1]).wait()
            pltpu.make_async_copy(e_hbm.at[pl.ds(base + c1 * CHUNK, CHUNK)],
                                  rows_v.at[1], esem[1]).wait()
            gather_start(1)
            pltpu.make_async_copy(rows_v.at[0], aggr_sh.at[id_v.at[0, 1]], ssem[0]).wait()
            ids_start(c0 + 2, 0)
            e_start(c0 + 2, 0)
            pltpu.make_async_copy(sd_hbm.at[wid, c0 + 2], id_v.at[0], isem[0]).wait()
            pltpu.make_async_copy(e_hbm.at[pl.ds(base + (c0 + 2) * CHUNK, CHUNK)],
                                  rows_v.at[0], esem[0]).wait()
            gather_start(0)
            pltpu.make_async_copy(h_hbm.at[id_v.at[1, 0]], rows_v.at[1], gsem[1]).wait()
            relu(1)
            scatter_start(1)
            pltpu.make_async_copy(rows_v.at[1], aggr_sh.at[id_v.at[1, 1]], ssem[1]).wait()
            c1n = jnp.minimum(c1 + 2, n_chunks - 1)
            ids_start(c1n, 1)
            e_start(c1n, 1)
            return 0
        lax.fori_loop(0, n_pairs, pair_body, 0)

        pltpu.make_async_copy(h_hbm.at[id_v.at[0, 0]], rows_v.at[0], gsem[0]).wait()
        relu(0)
        scatter_start(0)
        pltpu.make_async_copy(rows_v.at[0], aggr_sh.at[id_v.at[0, 1]], ssem[0]).wait()
        pltpu.make_async_copy(sd_hbm.at[wid, 0], id_v.at[1], isem[1]).wait()
        pltpu.make_async_copy(e_hbm.at[pl.ds(base, CHUNK)], rows_v.at[1], esem[1]).wait()

        plsc.subcore_barrier()
        pltpu.sync_copy(aggr_sh.at[pl.ds(row0, stripe)],
                        out_hbm.at[c, pl.ds(row0, stripe)])

    return sc_edge


def _node_body(h_ref, parts_ref, W1_ref, b1_ref, g1_ref, be1_ref, W2_ref,
               b2_ref, gbn_ref, bbn_ref, eps_ref, out_ref, *, final):
    pre = (1.0 + eps_ref[0, 0]) * h_ref[...] + parts_ref[0] + parts_ref[1]
    z = jnp.dot(pre, W1_ref[...], preferred_element_type=jnp.float32) + b1_ref[...]
    m = jnp.mean(z, axis=0, keepdims=True)
    v = jnp.mean((z - m) ** 2, axis=0, keepdims=True)
    z = (z - m) * lax.rsqrt(v + 1e-5) * g1_ref[...] + be1_ref[...]
    z = jax.nn.relu(z)
    hn = jnp.dot(z, W2_ref[...], preferred_element_type=jnp.float32) + b2_ref[...]
    m2 = jnp.mean(hn, axis=0, keepdims=True)
    v2 = jnp.mean((hn - m2) ** 2, axis=0, keepdims=True)
    hn = (hn - m2) * lax.rsqrt(v2 + 1e-5) * gbn_ref[...] + bbn_ref[...]
    if not final:
        hn = jax.nn.relu(hn)
    out_ref[...] = hn


def _node_stage(h, parts, W1, b1, g1, be1, W2, b2, gbn, bbn, eps, final):
    n, d = h.shape
    hdim = W1.shape[1]
    body = functools.partial(_node_body, final=final)
    return pl.pallas_call(
        body,
        grid=(1,),
        in_specs=[
            pl.BlockSpec((n, d), lambda i: (0, 0)),
            pl.BlockSpec((2, n, d), lambda i: (0, 0, 0)),
            pl.BlockSpec((d, hdim), lambda i: (0, 0)),
            pl.BlockSpec((1, hdim), lambda i: (0, 0)),
            pl.BlockSpec((1, hdim), lambda i: (0, 0)),
            pl.BlockSpec((1, hdim), lambda i: (0, 0)),
            pl.BlockSpec((hdim, d), lambda i: (0, 0)),
            pl.BlockSpec((1, d), lambda i: (0, 0)),
            pl.BlockSpec((1, d), lambda i: (0, 0)),
            pl.BlockSpec((1, d), lambda i: (0, 0)),
            pl.BlockSpec((1, 1), lambda i: (0, 0)),
        ],
        out_specs=pl.BlockSpec((n, d), lambda i: (0, 0)),
        out_shape=jax.ShapeDtypeStruct((n, d), jnp.float32),
    )(h, parts, W1, b1, g1, be1, W2, b2, gbn, bbn, eps)


def kernel(x, edge_index, edge_attr, W_edge, b_edge, W1, b1, g1, be1, W2, b2,
           g_bn, b_bn, eps_gin):
    n, d = x.shape
    n_edges = edge_attr.shape[0]
    n_layers = W_edge.shape[0]
    epw = n_edges // N_WORKERS
    assert n_edges % N_WORKERS == 0 and epw % CHUNK == 0 and n % 16 == 0

    src3 = edge_index[0].reshape(N_WORKERS, epw // CHUNK, CHUNK)
    dst3 = edge_index[1].reshape(N_WORKERS, epw // CHUNK, CHUNK)
    sd3 = jnp.stack([src3, dst3], axis=2)
    sc_edge = _make_sc_edge(n, d, n_edges)

    h = x
    for l in range(n_layers):
        e_l = _e_stage(edge_attr, W_edge[l], b_edge[l])
        parts = sc_edge(h, e_l, sd3)
        h = _node_stage(h, parts,
                        W1[l], b1[l].reshape(1, -1), g1[l].reshape(1, -1),
                        be1[l].reshape(1, -1), W2[l], b2[l].reshape(1, d),
                        g_bn[l].reshape(1, d), b_bn[l].reshape(1, d),
                        eps_gin[l].reshape(1, 1), final=(l == n_layers - 1))
    return h
